# initial kernel scaffold (unmeasured)
import jax
import jax.numpy as jnp
from jax import lax
from jax.experimental import pallas as pl
from jax.experimental.pallas import tpu as pltpu

N_DEV = 4


def kernel(x, w_mat):
    m_tot, k_blk = x.shape
    k_tot, n = w_mat.shape
    m_blk = m_tot // N_DEV

    def body(x_ref, w_ref, out_ref, comm_ref, send_sems, recv_sems):
        my = lax.axis_index("i")

        barrier_sem = pltpu.get_barrier_semaphore()
        for d in range(1, N_DEV):
            peer = (my + d) % N_DEV
            pl.semaphore_signal(
                barrier_sem, inc=1,
                device_id=(peer,), device_id_type=pl.DeviceIdType.MESH,
            )
        pl.semaphore_wait(barrier_sem, N_DEV - 1)

        rdmas = []
        for d in range(1, N_DEV):
            peer = (my + d) % N_DEV
            rdma = pltpu.make_async_remote_copy(
                src_ref=x_ref.at[pl.ds(peer * m_blk, m_blk), :],
                dst_ref=comm_ref.at[d - 1],
                send_sem=send_sems.at[d - 1],
                recv_sem=recv_sems.at[d - 1],
                device_id=(peer,),
                device_id_type=pl.DeviceIdType.MESH,
            )
            rdma.start()
            rdmas.append(rdma)

        out_ref[...] = jnp.dot(
            x_ref[pl.ds(my * m_blk, m_blk), :],
            w_ref[pl.ds(my * k_blk, k_blk), :],
            preferred_element_type=jnp.float32,
        )

        for d in range(1, N_DEV):
            rdmas[d - 1].wait_recv()
            k_origin = (my - d) % N_DEV
            out_ref[...] += jnp.dot(
                comm_ref[d - 1],
                w_ref[pl.ds(k_origin * k_blk, k_blk), :],
                preferred_element_type=jnp.float32,
            )

        out_ref[...] = jnp.maximum(out_ref[...], 0.0)

        for d in range(1, N_DEV):
            rdmas[d - 1].wait_send()

    return pl.pallas_call(
        body,
        out_shape=jax.ShapeDtypeStruct((m_blk, n), jnp.float32),
        in_specs=[
            pl.BlockSpec(memory_space=pltpu.VMEM),
            pl.BlockSpec(memory_space=pltpu.VMEM),
        ],
        out_specs=pl.BlockSpec(memory_space=pltpu.VMEM),
        scratch_shapes=[
            pltpu.VMEM((N_DEV - 1, m_blk, k_blk), x.dtype),
            pltpu.SemaphoreType.DMA((N_DEV - 1,)),
            pltpu.SemaphoreType.DMA((N_DEV - 1,)),
        ],
        compiler_params=pltpu.CompilerParams(collective_id=0),
    )(x, w_mat)


# baseline (device time: 107053 ns/iter reference)
import jax
import jax.numpy as jnp
from jax import lax
from jax.experimental import pallas as pl
from jax.experimental.pallas import tpu as pltpu

N_DEV = 4


def kernel(x, w_mat):
    x = x.astype(jnp.bfloat16)
    w_mat = w_mat.astype(jnp.bfloat16)
    m_tot, k_blk = x.shape
    k_tot, n = w_mat.shape
    m_blk = m_tot // N_DEV

    def body(x_ref, w_ref, out_ref, comm_ref, send_sems, recv_sems):
        my = lax.axis_index("i")

        barrier_sem = pltpu.get_barrier_semaphore()
        for d in range(1, N_DEV):
            peer = (my + d) % N_DEV
            pl.semaphore_signal(
                barrier_sem, inc=1,
                device_id=(peer,), device_id_type=pl.DeviceIdType.MESH,
            )
        pl.semaphore_wait(barrier_sem, N_DEV - 1)

        rdmas = []
        for d in range(1, N_DEV):
            peer = (my + d) % N_DEV
            rdma = pltpu.make_async_remote_copy(
                src_ref=x_ref.at[pl.ds(peer * m_blk, m_blk), :],
                dst_ref=comm_ref.at[d - 1],
                send_sem=send_sems.at[d - 1],
                recv_sem=recv_sems.at[d - 1],
                device_id=(peer,),
                device_id_type=pl.DeviceIdType.MESH,
            )
            rdma.start()
            rdmas.append(rdma)

        out_ref[...] = jnp.dot(
            x_ref[pl.ds(my * m_blk, m_blk), :],
            w_ref[pl.ds(my * k_blk, k_blk), :],
            preferred_element_type=jnp.float32,
        )

        for d in range(1, N_DEV):
            rdmas[d - 1].wait_recv()
            k_origin = (my - d) % N_DEV
            out_ref[...] += jnp.dot(
                comm_ref[d - 1],
                w_ref[pl.ds(k_origin * k_blk, k_blk), :],
                preferred_element_type=jnp.float32,
            )

        out_ref[...] = jnp.maximum(out_ref[...], 0.0)

        for d in range(1, N_DEV):
            rdmas[d - 1].wait_send()

    return pl.pallas_call(
        body,
        out_shape=jax.ShapeDtypeStruct((m_blk, n), jnp.float32),
        in_specs=[
            pl.BlockSpec(memory_space=pltpu.VMEM),
            pl.BlockSpec(memory_space=pltpu.VMEM),
        ],
        out_specs=pl.BlockSpec(memory_space=pltpu.VMEM),
        scratch_shapes=[
            pltpu.VMEM((N_DEV - 1, m_blk, k_blk), x.dtype),
            pltpu.SemaphoreType.DMA((N_DEV - 1,)),
            pltpu.SemaphoreType.DMA((N_DEV - 1,)),
        ],
        compiler_params=pltpu.CompilerParams(
            collective_id=0,
            vmem_limit_bytes=100 * 1024 * 1024,
        ),
    )(x, w_mat)


# device time: 77828 ns/iter; 1.3755x vs baseline; 1.3755x over previous
import jax
import jax.numpy as jnp
from jax import lax
from jax.experimental import pallas as pl
from jax.experimental.pallas import tpu as pltpu

N_DEV = 4


def kernel(x, w_mat):
    m_tot, k_blk = x.shape
    k_tot, n = w_mat.shape
    m_blk = m_tot // N_DEV

    wait_order = (1, 3, 2)

    def body(x_ref, w_ref, out_ref, xb_ref, wb_ref, comm_ref, xst_ref,
             wst_ref, xdma_sems, wdma_sem, send_sems, recv_sems):
        my = lax.axis_index("i")

        barrier_sem = pltpu.get_barrier_semaphore()
        for d in range(1, N_DEV):
            peer = (my + d) % N_DEV
            pl.semaphore_signal(
                barrier_sem, inc=1,
                device_id=(peer,), device_id_type=pl.DeviceIdType.MESH,
            )
        pl.semaphore_wait(barrier_sem, N_DEV - 1)

        m_blocks = [(my + d) % N_DEV for d in range(1, N_DEV)] + [my]

        def x_dma(idx, slot):
            return pltpu.make_async_copy(
                x_ref.at[pl.ds(m_blocks[idx] * m_blk, m_blk), :],
                xst_ref.at[slot],
                xdma_sems.at[slot],
            )

        x_dma(0, 0).start()
        rdmas = [None] * (N_DEV - 1)
        for idx in range(N_DEV):
            slot = idx % 2
            x_dma(idx, slot).wait()
            if idx + 1 < N_DEV:
                x_dma(idx + 1, 1 - slot).start()
            xb_ref[idx] = xst_ref[slot].astype(jnp.bfloat16)
            if idx < N_DEV - 1:
                rdma = pltpu.make_async_remote_copy(
                    src_ref=xb_ref.at[idx],
                    dst_ref=comm_ref.at[idx],
                    send_sem=send_sems.at[idx],
                    recv_sem=recv_sems.at[idx],
                    device_id=(m_blocks[idx],),
                    device_id_type=pl.DeviceIdType.MESH,
                )
                rdma.start()
                rdmas[idx] = rdma

        k_blocks = [my] + [(my - d) % N_DEV for d in wait_order]

        def w_dma(j):
            return pltpu.make_async_copy(
                w_ref.at[pl.ds(k_blocks[j] * k_blk, k_blk), :],
                wst_ref,
                wdma_sem,
            )

        w_dma(0).start()
        w_dma(0).wait()
        wb_ref[0] = wst_ref[...].astype(jnp.bfloat16)

        out_ref[...] = jnp.dot(
            xb_ref[N_DEV - 1], wb_ref[0],
            preferred_element_type=jnp.float32,
        )

        for j, d in enumerate(wait_order):
            w_dma(j + 1).start()
            w_dma(j + 1).wait()
            wb_ref[j + 1] = wst_ref[...].astype(jnp.bfloat16)
            rdmas[d - 1].wait_recv()
            out_ref[...] += jnp.dot(
                comm_ref[d - 1], wb_ref[j + 1],
                preferred_element_type=jnp.float32,
            )

        out_ref[...] = jnp.maximum(out_ref[...], 0.0)

        for d in range(1, N_DEV):
            rdmas[d - 1].wait_send()

    return pl.pallas_call(
        body,
        out_shape=jax.ShapeDtypeStruct((m_blk, n), jnp.float32),
        in_specs=[
            pl.BlockSpec(memory_space=pl.ANY),
            pl.BlockSpec(memory_space=pl.ANY),
        ],
        out_specs=pl.BlockSpec(memory_space=pltpu.VMEM),
        scratch_shapes=[
            pltpu.VMEM((N_DEV, m_blk, k_blk), jnp.bfloat16),
            pltpu.VMEM((N_DEV, k_blk, n), jnp.bfloat16),
            pltpu.VMEM((N_DEV - 1, m_blk, k_blk), jnp.bfloat16),
            pltpu.VMEM((2, m_blk, k_blk), jnp.float32),
            pltpu.VMEM((k_blk, n), jnp.float32),
            pltpu.SemaphoreType.DMA((2,)),
            pltpu.SemaphoreType.DMA,
            pltpu.SemaphoreType.DMA((N_DEV - 1,)),
            pltpu.SemaphoreType.DMA((N_DEV - 1,)),
        ],
        compiler_params=pltpu.CompilerParams(
            collective_id=0,
            vmem_limit_bytes=63 * 1024 * 1024,
        ),
    )(x, w_mat)


# device time: 74162 ns/iter; 1.4435x vs baseline; 1.0494x over previous
import jax
import jax.numpy as jnp
from jax import lax
from jax.experimental import pallas as pl
from jax.experimental.pallas import tpu as pltpu

N_DEV = 4
MSGS = ((0, 0, 512), (0, 512, 512), (1, 0, 1024), (2, 0, 512), (2, 512, 512))


def kernel(x, w_mat):
    m_tot, k_blk = x.shape
    k_tot, n = w_mat.shape
    m_blk = m_tot // N_DEV

    def body(x_ref, w_ref, out_ref, xb_ref, wb_ref, comm_ref, xst_ref,
             wst_ref, xdma_sems, wdma_sem, send_sems, recv_sems):
        my = lax.axis_index("i")

        m_blocks = [(my + d) % N_DEV for d in range(1, N_DEV)] + [my]

        def x_dma(idx, slot, sem, off=0, rows=m_blk):
            return pltpu.make_async_copy(
                x_ref.at[pl.ds(m_blocks[idx] * m_blk + off, rows), :],
                xst_ref.at[slot, pl.ds(off, rows), :],
                xdma_sems.at[sem],
            )

        dma_a = x_dma(0, 0, 0, 0, m_blk // 2)
        dma_b = x_dma(0, 0, 2, m_blk // 2, m_blk // 2)
        dma_a.start()
        dma_b.start()

        barrier_sem = pltpu.get_barrier_semaphore()
        for d in range(1, N_DEV):
            peer = (my + d) % N_DEV
            pl.semaphore_signal(
                barrier_sem, inc=1,
                device_id=(peer,), device_id_type=pl.DeviceIdType.MESH,
            )
        pl.semaphore_wait(barrier_sem, N_DEV - 1)

        def send(msg, idx):
            t, off, rows = MSGS[msg]
            rdma = pltpu.make_async_remote_copy(
                src_ref=xb_ref.at[idx, pl.ds(off, rows), :],
                dst_ref=comm_ref.at[t, pl.ds(off, rows), :],
                send_sem=send_sems.at[msg],
                recv_sem=recv_sems.at[msg],
                device_id=(m_blocks[idx],),
                device_id_type=pl.DeviceIdType.MESH,
            )
            rdma.start()
            return rdma

        rdmas = [None] * len(MSGS)
        half = pl.ds(0, m_blk // 2)
        dma_a.wait()
        xb_ref[0, half, :] = xst_ref[0, half, :].astype(jnp.bfloat16)
        rdmas[0] = send(0, 0)
        dma_c = x_dma(1, 1, 1)
        dma_c.start()
        half2 = pl.ds(m_blk // 2, m_blk // 2)
        dma_b.wait()
        xb_ref[0, half2, :] = xst_ref[0, half2, :].astype(jnp.bfloat16)
        rdmas[1] = send(1, 0)
        dma_c.wait()
        xb_ref[1] = xst_ref[1].astype(jnp.bfloat16)
        rdmas[2] = send(2, 1)
        dma_d = x_dma(2, 0, 0)
        dma_d.start()
        dma_e = x_dma(3, 1, 2)
        dma_e.start()
        dma_d.wait()
        xb_ref[2] = xst_ref[0].astype(jnp.bfloat16)
        rdmas[3] = send(3, 2)
        rdmas[4] = send(4, 2)
        dma_e.wait()
        xb_ref[3] = xst_ref[1].astype(jnp.bfloat16)

        k_blocks = [my] + [(my - d) % N_DEV for d in range(1, N_DEV)]

        def w_dma(j):
            return pltpu.make_async_copy(
                w_ref.at[pl.ds(k_blocks[j] * k_blk, k_blk), :],
                wst_ref,
                wdma_sem,
            )

        w_dma(0).start()
        w_dma(0).wait()
        wb_ref[0] = wst_ref[...].astype(jnp.bfloat16)

        out_ref[...] = jnp.dot(
            xb_ref[N_DEV - 1], wb_ref[0],
            preferred_element_type=jnp.float32,
        )

        msg = 0
        for j in range(1, N_DEV):
            w_dma(j).start()
            w_dma(j).wait()
            wb_ref[j] = wst_ref[...].astype(jnp.bfloat16)
            while msg < len(MSGS) and MSGS[msg][0] == j - 1:
                t, off, nrows = MSGS[msg]
                rdmas[msg].wait_recv()
                rows = pl.ds(off, nrows)
                out_ref[rows, :] += jnp.dot(
                    comm_ref[t, rows, :], wb_ref[j],
                    preferred_element_type=jnp.float32,
                )
                if j == N_DEV - 1:
                    out_ref[rows, :] = jnp.maximum(out_ref[rows, :], 0.0)
                msg += 1

        for rdma in rdmas:
            rdma.wait_send()

    return pl.pallas_call(
        body,
        out_shape=jax.ShapeDtypeStruct((m_blk, n), jnp.float32),
        in_specs=[
            pl.BlockSpec(memory_space=pl.ANY),
            pl.BlockSpec(memory_space=pl.ANY),
        ],
        out_specs=pl.BlockSpec(memory_space=pltpu.VMEM),
        scratch_shapes=[
            pltpu.VMEM((N_DEV, m_blk, k_blk), jnp.bfloat16),
            pltpu.VMEM((N_DEV, k_blk, n), jnp.bfloat16),
            pltpu.VMEM((N_DEV - 1, m_blk, k_blk), jnp.bfloat16),
            pltpu.VMEM((2, m_blk, k_blk), jnp.float32),
            pltpu.VMEM((k_blk, n), jnp.float32),
            pltpu.SemaphoreType.DMA((3,)),
            pltpu.SemaphoreType.DMA,
            pltpu.SemaphoreType.DMA((len(MSGS),)),
            pltpu.SemaphoreType.DMA((len(MSGS),)),
        ],
        compiler_params=pltpu.CompilerParams(
            collective_id=0,
            vmem_limit_bytes=63 * 1024 * 1024,
        ),
    )(x, w_mat)
